# Initial kernel scaffold; baseline (speedup 1.0000x reference)
import jax
import jax.numpy as jnp
from jax.experimental import pallas as pl


def kernel(cell_x, edge_index, edge_attr, W1, b1, W2, b2, U1, c1, U2, c2, gamma, beta):
    raise NotImplementedError("write your pallas kernel here")


# speedup vs baseline: 458.0199x; 458.0199x over previous
import jax
import jax.numpy as jnp
from jax import lax
from jax.experimental import pallas as pl
from jax.experimental.pallas import tpu as pltpu
from jax.experimental.pallas import tpu_sc as plsc

N, E, HID, EDIM = 10000, 320000, 128, 16
NC, NS = 2, 16
NW = NC * NS
SB = 80
ROWS = 128
EPW = ROWS * SB
EPAD = NW * EPW
GRP = 8
NGRP = ROWS // GRP
NPAD = 10240
DUMMY = N + 8
RPT = NPAD // NS
NP8 = RPT // SB


CROWS = NPAD // HID


def _sc_body(npre, eap, src3, dst3, out_s,
             s_sp, rows0, rows1, acc0, acc1,
             idxs_a, idxd_a, idxs_b, idxd_b, g_sem, e_sem, s_sem):
    c = lax.axis_index("c")
    s = lax.axis_index("s")
    w = c * NS + s
    zero16 = jnp.zeros((16,), jnp.float32)
    rows = (rows0, rows1)
    acc = (acc0, acc1)
    idxs = (idxs_a, idxs_b)
    idxd = (idxd_a, idxd_b)

    def zrow(i, carry):
        for g in range(HID // 16):
            sl = pl.ds(g * 16, 16)
            acc0[i, sl] = zero16
            acc1[i, sl] = zero16
        return carry
    lax.fori_loop(0, SB, zrow, 0)

    for p in range(NP8):
        pltpu.sync_copy(acc0, s_sp.at[pl.ds(s * RPT + p * SB, SB)])
    plsc.subcore_barrier()

    pltpu.sync_copy(src3.at[w, pl.ds(0, GRP)], idxs_a)
    pltpu.sync_copy(dst3.at[w, pl.ds(0, GRP)], idxd_a)
    pltpu.async_copy(npre.at[idxs_a.at[0]], rows0, g_sem)
    pltpu.async_copy(eap.at[pl.ds(w * EPW, SB)], acc0, e_sem)
    pltpu.async_copy(acc1, s_sp.at[idxd_a.at[0]], s_sem, add=True)

    def two_groups(m, carry):
        for half in range(2):
            k = 2 * m + half
            cs, cd = idxs[half], idxd[half]
            ns, nd = idxs[1 - half], idxd[1 - half]
            knxt = lax.rem(k + 1, NGRP)
            pltpu.sync_copy(src3.at[w, pl.ds(knxt * GRP, GRP)], ns)
            pltpu.sync_copy(dst3.at[w, pl.ds(knxt * GRP, GRP)], nd)
            for t in range(GRP):
                p = t % 2
                off = w * EPW + (k * GRP + t) * SB
                pltpu.make_async_copy(npre.at[cs.at[t]], rows[p],
                                      g_sem).wait()
                pltpu.make_async_copy(eap.at[pl.ds(off, SB)], acc[p],
                                      e_sem).wait()

                def comp(i, carry2):
                    for g in range(HID // 16):
                        sl = pl.ds(g * 16, 16)
                        acc[p][i, sl] = jnp.maximum(
                            acc[p][i, sl] + rows[p][i, sl], 0.0)
                    return carry2
                lax.fori_loop(0, SB, comp, 0)

                nref = cs.at[t + 1] if t < GRP - 1 else ns.at[0]
                pltpu.async_copy(npre.at[nref], rows[1 - p], g_sem)
                pltpu.async_copy(acc[p], s_sp.at[cd.at[t]], s_sem, add=True)
                pltpu.make_async_copy(acc[1 - p], s_sp.at[cd.at[t]],
                                      s_sem).wait()
                offn = jnp.minimum(off + SB, (w + 1) * EPW - SB)
                pltpu.async_copy(eap.at[pl.ds(offn, SB)], acc[1 - p], e_sem)
        return carry
    lax.fori_loop(0, NGRP // 2, two_groups, 0)

    pltpu.make_async_copy(npre.at[idxs_a.at[0]], rows[0], g_sem).wait()
    pltpu.make_async_copy(eap.at[pl.ds(w * EPW, SB)], acc[1], e_sem).wait()
    pltpu.make_async_copy(acc[1], s_sp.at[idxd_a.at[0]], s_sem).wait()

    plsc.subcore_barrier()
    for p in range(NP8):
        r0 = s * RPT + p * SB
        pltpu.sync_copy(s_sp.at[pl.ds(r0, SB)], acc0)
        pltpu.sync_copy(acc0, out_s.at[c, pl.ds(r0, SB)])


def _make_sc_agg():
    return pl.kernel(
        _sc_body,
    out_type=jax.ShapeDtypeStruct((NC, NPAD, HID), jnp.float32),
    mesh=plsc.VectorSubcoreMesh(core_axis_name="c", subcore_axis_name="s"),
    scratch_types=[
        pltpu.VMEM_SHARED((NPAD, HID), jnp.float32),
        pltpu.VMEM((SB, HID), jnp.float32),
        pltpu.VMEM((SB, HID), jnp.float32),
        pltpu.VMEM((SB, HID), jnp.float32),
        pltpu.VMEM((SB, HID), jnp.float32),
        pltpu.VMEM((GRP, SB), jnp.int32),
        pltpu.VMEM((GRP, SB), jnp.int32),
        pltpu.VMEM((GRP, SB), jnp.int32),
        pltpu.VMEM((GRP, SB), jnp.int32),
        pltpu.SemaphoreType.DMA,
        pltpu.SemaphoreType.DMA,
        pltpu.SemaphoreType.DMA,
    ],
)


_SC_AGG_CACHE = []


def _sc_agg(*args):
    if not _SC_AGG_CACHE:
        _SC_AGG_CACHE.append(_make_sc_agg())
    return _SC_AGG_CACHE[0](*args)


def _npre_body(x_ref, w_ref, b_ref, o_ref):
    o_ref[...] = jnp.dot(x_ref[...], w_ref[...],
                         preferred_element_type=jnp.float32) + b_ref[...]


def _node_pre(x, w1a, b1):
    blk = 2000
    return pl.pallas_call(
        _npre_body,
        grid=(N // blk,),
        in_specs=[pl.BlockSpec((blk, HID), lambda i: (i, 0)),
                  pl.BlockSpec((HID, HID), lambda i: (0, 0)),
                  pl.BlockSpec((1, HID), lambda i: (0, 0))],
        out_specs=pl.BlockSpec((blk, HID), lambda i: (i, 0)),
        out_shape=jax.ShapeDtypeStruct((N, HID), jnp.float32),
    )(x, w1a, b1)


def _eap_body(a_ref, w_ref, o_ref):
    o_ref[...] = jnp.dot(a_ref[...], w_ref[...],
                         preferred_element_type=jnp.float32)


def _ea_part(ea, w1b):
    blk = 2048
    return pl.pallas_call(
        _eap_body,
        grid=(EPAD // blk,),
        in_specs=[pl.BlockSpec((blk, EDIM), lambda i: (i, 0)),
                  pl.BlockSpec((EDIM, HID), lambda i: (0, 0))],
        out_specs=pl.BlockSpec((blk, HID), lambda i: (i, 0)),
        out_shape=jax.ShapeDtypeStruct((EPAD, HID), jnp.float32),
    )(ea, w1b)


def _chist_body(d_ref, o_ref):
    i = pl.program_id(0)
    d = d_ref[...]
    iot = jax.lax.broadcasted_iota(jnp.int32, (1, HID), 1)
    a = (lax.shift_right_logical(d, 7) == iot).astype(jnp.float32)
    b = (lax.bitwise_and(d, 127) == iot).astype(jnp.float32)
    c = lax.dot_general(a, b, (((0,), (0,)), ((), ())),
                        preferred_element_type=jnp.float32)

    @pl.when(i == 0)
    def _():
        o_ref[...] = jnp.zeros_like(o_ref)
    o_ref[...] += c


def _cnt_hist(dst_pad):
    blk = 2048
    return pl.pallas_call(
        _chist_body,
        grid=(EPAD // blk,),
        in_specs=[pl.BlockSpec((blk, 1), lambda i: (i, 0))],
        out_specs=pl.BlockSpec((HID, HID), lambda i: (0, 0)),
        out_shape=jax.ShapeDtypeStruct((HID, HID), jnp.float32),
    )(dst_pad)


def _post_body(x_ref, s0_ref, s1_ref, cnt_ref, w2_ref, b2_ref,
               u1a_ref, u1b_ref, c1v_ref, u2_ref, c2v_ref, g_ref, be_ref,
               o_ref):
    x = x_ref[...]
    s = s0_ref[0] + s1_ref[0]
    cnt = cnt_ref[...]
    aggw = jnp.dot(s, w2_ref[...], preferred_element_type=jnp.float32)
    agg = (aggw + cnt * b2_ref[...]) / jnp.maximum(cnt, 1.0)
    t = jnp.dot(x, u1a_ref[...], preferred_element_type=jnp.float32)
    t += jnp.dot(agg, u1b_ref[...], preferred_element_type=jnp.float32)
    t = jnp.maximum(t + c1v_ref[...], 0.0)
    out = jnp.dot(t, u2_ref[...], preferred_element_type=jnp.float32)
    y = x + out + c2v_ref[...]
    mean = jnp.mean(y, axis=-1, keepdims=True)
    var = jnp.mean((y - mean) ** 2, axis=-1, keepdims=True)
    o_ref[...] = (y - mean) * lax.rsqrt(var + 1e-5) * g_ref[...] + be_ref[...]


def _post(x, s_part, cnt, w2, b2, u1a, u1b, c1v, u2, c2v, gamma, beta):
    blk = 2000
    row = lambda i: (i, 0)
    full = lambda i: (0, 0)
    return pl.pallas_call(
        _post_body,
        grid=(N // blk,),
        in_specs=[pl.BlockSpec((blk, HID), row),
                  pl.BlockSpec((1, blk, HID), lambda i: (0, i, 0)),
                  pl.BlockSpec((1, blk, HID), lambda i: (1, i, 0)),
                  pl.BlockSpec((blk, 1), row),
                  pl.BlockSpec((HID, HID), full),
                  pl.BlockSpec((1, HID), full),
                  pl.BlockSpec((HID, HID), full),
                  pl.BlockSpec((HID, HID), full),
                  pl.BlockSpec((1, HID), full),
                  pl.BlockSpec((HID, HID), full),
                  pl.BlockSpec((1, HID), full),
                  pl.BlockSpec((1, HID), full),
                  pl.BlockSpec((1, HID), full)],
        out_specs=pl.BlockSpec((blk, HID), row),
        out_shape=jax.ShapeDtypeStruct((N, HID), jnp.float32),
    )(x, s_part, s_part, cnt, w2, b2, u1a, u1b, c1v, u2, c2v, gamma, beta)


def kernel(cell_x, edge_index, edge_attr, W1, b1, W2, b2, U1, c1, U2, c2,
           gamma, beta):
    x = cell_x[0]
    ea = edge_attr[0]
    src = jnp.clip(edge_index[0, :, 0], 0, N - 1).astype(jnp.int32)
    dst = jnp.clip(edge_index[0, :, 1], 0, N - 1).astype(jnp.int32)
    ppw = EPW - E // NW
    dpad = jnp.broadcast_to(N + (jnp.arange(ppw, dtype=jnp.int32)
                                 % (NPAD - N)), (NW, ppw))
    src3 = jnp.concatenate(
        [src.reshape(NW, E // NW), jnp.zeros((NW, ppw), jnp.int32)],
        axis=1).reshape(NW, ROWS, SB)
    dst_pad = jnp.concatenate([dst.reshape(NW, E // NW), dpad], axis=1)
    dst3 = dst_pad.reshape(NW, ROWS, SB)
    ea_pad = jnp.concatenate(
        [ea.reshape(NW, E // NW, EDIM),
         jnp.zeros((NW, ppw, EDIM), jnp.float32)], axis=1).reshape(EPAD,
                                                                   EDIM)
    npre = _node_pre(x, W1[:HID], b1.reshape(1, HID))
    eap = _ea_part(ea_pad, W1[HID:])
    s_part = _sc_agg(npre, eap, src3, dst3)
    cnt = _cnt_hist(dst_pad.reshape(EPAD, 1))
    cnt = cnt[:NPAD // HID].reshape(NPAD)[:N, None]
    out = _post(x, s_part, cnt,
                W2, b2.reshape(1, HID), U1[:HID], U1[HID:],
                c1.reshape(1, HID), U2, c2.reshape(1, HID),
                gamma.reshape(1, HID), beta.reshape(1, HID))
    return out[None]
